# baseline (speedup 1.0000x reference)
import jax
import jax.numpy as jnp
from jax import lax
from jax.experimental import pallas as pl
from jax.experimental.pallas import tpu as pltpu
from jax.experimental.pallas import tpu_sc as plsc

STEP = 8
NROWS = 16
SEQ = 2048
D = 512
MAXLEN = SEQ - 1
NOUT = MAXLEN // STEP + 1
TOTAL = NROWS * NOUT
NWORK = 32
CHUNK = TOTAL // NWORK
LANES = 16


def _flat_src(p, lens_v):
    b = lax.shift_right_logical(p, 8)
    k = p - b * NOUT
    l = plsc.load_gather(lens_v, [b])
    start = jnp.minimum(l - 1, (STEP - 1) + (l & (STEP - 1)))
    v = start + (k - 1) * STEP
    idx = jnp.where(k == 0, 0, jnp.where(v < l, v, MAXLEN - 1) + 1)
    return b * SEQ + idx


NCH = 2
SUB = CHUNK // NCH


def _body(payload_hbm, lens_hbm, out_hbm, olens_hbm,
          lens_v, idx_v, rows0_v, rows1_v, olens_v, semg0, semg1, sems0, sems1):
    wid = lax.axis_index("s") * 2 + lax.axis_index("c")
    pltpu.sync_copy(lens_hbm, lens_v)
    iota = lax.iota(jnp.int32, LANES)

    base = wid * CHUNK
    for c in range(NCH):
        for g in range(SUB // LANES):
            idx_v[c, pl.ds(g * LANES, LANES)] = _flat_src(
                base + c * SUB + g * LANES + iota, lens_v)

    bufs = (rows0_v, rows1_v)
    semg = (semg0, semg1)
    sems = (sems0, sems1)

    def gather(c):
        return pltpu.async_copy(payload_hbm.at[idx_v.at[c]],
                                bufs[c % 2], semg[c % 2])

    def put(c):
        return pltpu.async_copy(bufs[c % 2],
                                out_hbm.at[pl.ds(base + c * SUB, SUB)],
                                sems[c % 2])

    descs = [None] * NCH
    descs[0] = gather(0)
    puts = [None] * NCH
    for c in range(NCH):
        if c + 1 < NCH:
            if c >= 1:
                puts[c - 1].wait()
            descs[c + 1] = gather(c + 1)
        descs[c].wait()
        puts[c] = put(c)

    @pl.when(wid == 0)
    def _olens():
        olens_v[...] = jnp.minimum(1, lax.shift_right_logical(lens_v[...], 3))
        pltpu.sync_copy(olens_v, olens_hbm)

    puts[NCH - 2].wait()
    puts[NCH - 1].wait()


_sc_call = pl.kernel(
    _body,
    out_type=(
        jax.ShapeDtypeStruct((TOTAL, D), jnp.float32),
        jax.ShapeDtypeStruct((NROWS,), jnp.int32),
    ),
    mesh=plsc.VectorSubcoreMesh(core_axis_name="c", subcore_axis_name="s"),
    compiler_params=pltpu.CompilerParams(needs_layout_passes=False),
    scratch_types=(
        pltpu.VMEM((NROWS,), jnp.int32),
        pltpu.VMEM((NCH, SUB), jnp.int32),
        pltpu.VMEM((SUB, D), jnp.float32),
        pltpu.VMEM((SUB, D), jnp.float32),
        pltpu.VMEM((NROWS,), jnp.int32),
        pltpu.SemaphoreType.DMA,
        pltpu.SemaphoreType.DMA,
        pltpu.SemaphoreType.DMA,
        pltpu.SemaphoreType.DMA,
    ),
)


@jax.jit
def kernel(x_payload, x_seq_lens):
    flat = x_payload.reshape(NROWS * SEQ, D)
    out, out_lens = _sc_call(flat, x_seq_lens.astype(jnp.int32))
    return out.reshape(NROWS, NOUT, D), out_lens


# speedup vs baseline: 1.0263x; 1.0263x over previous
import jax
import jax.numpy as jnp
from jax import lax
from jax.experimental import pallas as pl
from jax.experimental.pallas import tpu as pltpu
from jax.experimental.pallas import tpu_sc as plsc

STEP = 8
NROWS = 16
SEQ = 2048
D = 512
MAXLEN = SEQ - 1
NOUT = MAXLEN // STEP + 1
TOTAL = NROWS * NOUT
NWORK = 32
CHUNK = TOTAL // NWORK
LANES = 16


def _flat_src(p, lens_v):
    b = lax.shift_right_logical(p, 8)
    k = p - b * NOUT
    l = plsc.load_gather(lens_v, [b])
    start = jnp.minimum(l - 1, (STEP - 1) + (l & (STEP - 1)))
    v = start + (k - 1) * STEP
    idx = jnp.where(k == 0, 0, jnp.where(v < l, v, MAXLEN - 1) + 1)
    return b * SEQ + idx


NCH = 8
SUB = CHUNK // NCH


def _body(payload_hbm, lens_hbm, out_hbm, olens_hbm,
          lens_v, idx_v, rows0_v, rows1_v, olens_v, semg0, semg1, sems0, sems1):
    wid = lax.axis_index("s") * 2 + lax.axis_index("c")
    pltpu.sync_copy(lens_hbm, lens_v)
    iota = lax.iota(jnp.int32, LANES)

    base = wid * CHUNK
    for c in range(NCH):
        for g in range(SUB // LANES):
            idx_v[c, pl.ds(g * LANES, LANES)] = _flat_src(
                base + c * SUB + g * LANES + iota, lens_v)

    bufs = (rows0_v, rows1_v)
    semg = (semg0, semg1)
    sems = (sems0, sems1)

    def gather(c):
        return pltpu.async_copy(payload_hbm.at[idx_v.at[c]],
                                bufs[c % 2], semg[c % 2])

    def put(c):
        return pltpu.async_copy(bufs[c % 2],
                                out_hbm.at[pl.ds(base + c * SUB, SUB)],
                                sems[c % 2])

    descs = [None] * NCH
    descs[0] = gather(0)
    puts = [None] * NCH
    for c in range(NCH):
        if c + 1 < NCH:
            if c >= 1:
                puts[c - 1].wait()
            descs[c + 1] = gather(c + 1)
        descs[c].wait()
        puts[c] = put(c)

    @pl.when(wid == 0)
    def _olens():
        olens_v[...] = jnp.minimum(1, lax.shift_right_logical(lens_v[...], 3))
        pltpu.sync_copy(olens_v, olens_hbm)

    puts[NCH - 2].wait()
    puts[NCH - 1].wait()


_sc_call = pl.kernel(
    _body,
    out_type=(
        jax.ShapeDtypeStruct((TOTAL, D), jnp.float32),
        jax.ShapeDtypeStruct((NROWS,), jnp.int32),
    ),
    mesh=plsc.VectorSubcoreMesh(core_axis_name="c", subcore_axis_name="s"),
    compiler_params=pltpu.CompilerParams(needs_layout_passes=False),
    scratch_types=(
        pltpu.VMEM((NROWS,), jnp.int32),
        pltpu.VMEM((NCH, SUB), jnp.int32),
        pltpu.VMEM((SUB, D), jnp.float32),
        pltpu.VMEM((SUB, D), jnp.float32),
        pltpu.VMEM((NROWS,), jnp.int32),
        pltpu.SemaphoreType.DMA,
        pltpu.SemaphoreType.DMA,
        pltpu.SemaphoreType.DMA,
        pltpu.SemaphoreType.DMA,
    ),
)


@jax.jit
def kernel(x_payload, x_seq_lens):
    flat = x_payload.reshape(NROWS * SEQ, D)
    out, out_lens = _sc_call(flat, x_seq_lens.astype(jnp.int32))
    return out.reshape(NROWS, NOUT, D), out_lens


# speedup vs baseline: 1.5001x; 1.4617x over previous
import jax
import jax.numpy as jnp
from jax import lax
from jax.experimental import pallas as pl
from jax.experimental.pallas import tpu as pltpu
from jax.experimental.pallas import tpu_sc as plsc

STEP = 8
NROWS = 16
SEQ = 2048
D = 512
MAXLEN = SEQ - 1
NOUT = MAXLEN // STEP + 1
TOTAL = NROWS * NOUT
NWORK = 32
CHUNK = TOTAL // NWORK
LANES = 16
NCH = 4
SUB = CHUNK // NCH
PSRC = 16


def _flat_src(p, lens_v):
    b = lax.shift_right_logical(p, 8)
    k = p - b * NOUT
    l = plsc.load_gather(lens_v, [b])
    start = jnp.minimum(l - 1, (STEP - 1) + (l & (STEP - 1)))
    v = start + (k - 1) * STEP
    idx = jnp.where(k == 0, 0, jnp.where(v < l, v, MAXLEN - 1) + 1)
    return b * SEQ + idx


def _body(payload_hbm, lens_hbm, out_hbm, olens_hbm,
          lens_v, idx_v, pidx_v, buf_v, psrc_v, olens_v,
          semg0, semg1, semg2, semg3, semp, semp0, semp1, semp2, semp3):
    wid = lax.axis_index("s") * 2 + lax.axis_index("c")
    pltpu.sync_copy(lens_hbm, lens_v)
    iota = lax.iota(jnp.int32, LANES)
    base = wid * CHUNK
    b = lax.shift_right_logical(wid, 1)
    h = wid & 1

    pidx_v[...] = (b * SEQ + SEQ - PSRC) + iota
    psd = pltpu.async_copy(payload_hbm.at[pidx_v], psrc_v, semp)

    l = jnp.max(plsc.load_gather(lens_v, [iota * 0 + b]))
    bv = jnp.minimum(jnp.maximum(lax.shift_right_logical(l, 3), 1), NOUT - 1)
    fp = jnp.where(h == 0, 1 + jnp.minimum(bv, CHUNK - 1),
                   jnp.clip(bv - (CHUNK - 1), 0, CHUNK))

    for c in range(NCH):
        for g in range(SUB // LANES):
            s = c * SUB + g * LANES + iota
            real = _flat_src(base + s, lens_v)
            dummy = base * (SEQ // NOUT) + s
            idx_v[c, pl.ds(g * LANES, LANES)] = jnp.where(s < fp, real, dummy)

    semg = (semg0, semg1, semg2, semg3)
    gd = [pltpu.async_copy(payload_hbm.at[idx_v.at[t]],
                           buf_v.at[pl.ds(t * SUB, SUB)], semg[t])
          for t in range(NCH)]

    psd.wait()
    padv = [psrc_v[PSRC - 1, pl.ds(i * LANES, LANES)]
            for i in range(D // LANES)]

    dsts = [out_hbm.at[pl.ds(base + t * SUB, SUB)] for t in range(NCH)]
    semput = (semp0, semp1, semp2, semp3)
    pds = [None] * NCH
    for t in range(NCH):
        gd[t].wait()

        def _fill(rr, carry, t=t):
            for i in range(D // LANES):
                buf_v[t * SUB + rr, pl.ds(i * LANES, LANES)] = padv[i]
            return carry

        lo = jnp.clip(fp - t * SUB, 0, SUB)
        lax.fori_loop(lo, SUB, _fill, 0)

        pds[t] = pltpu.async_copy(buf_v.at[pl.ds(t * SUB, SUB)],
                                  dsts[t], semput[t])

    @pl.when(wid == 0)
    def _olens():
        olens_v[...] = jnp.minimum(1, lax.shift_right_logical(lens_v[...], 3))
        pltpu.sync_copy(olens_v, olens_hbm)

    for t in range(NCH):
        pds[t].wait()


_sc_call = pl.kernel(
    _body,
    out_type=(
        jax.ShapeDtypeStruct((TOTAL, D), jnp.float32),
        jax.ShapeDtypeStruct((NROWS,), jnp.int32),
    ),
    mesh=plsc.VectorSubcoreMesh(core_axis_name="c", subcore_axis_name="s"),
    compiler_params=pltpu.CompilerParams(needs_layout_passes=False),
    scratch_types=(
        pltpu.VMEM((NROWS,), jnp.int32),
        pltpu.VMEM((NCH, SUB), jnp.int32),
        pltpu.VMEM((LANES,), jnp.int32),
        pltpu.VMEM((CHUNK, D), jnp.float32),
        pltpu.VMEM((PSRC, D), jnp.float32),
        pltpu.VMEM((NROWS,), jnp.int32),
        pltpu.SemaphoreType.DMA,
        pltpu.SemaphoreType.DMA,
        pltpu.SemaphoreType.DMA,
        pltpu.SemaphoreType.DMA,
        pltpu.SemaphoreType.DMA,
        pltpu.SemaphoreType.DMA,
        pltpu.SemaphoreType.DMA,
        pltpu.SemaphoreType.DMA,
        pltpu.SemaphoreType.DMA,
    ),
)


@jax.jit
def kernel(x_payload, x_seq_lens):
    flat = x_payload.reshape(NROWS * SEQ, D)
    out, out_lens = _sc_call(flat, x_seq_lens.astype(jnp.int32))
    return out.reshape(NROWS, NOUT, D), out_lens


# speedup vs baseline: 1.5087x; 1.0057x over previous
import jax
import jax.numpy as jnp
from jax import lax
from jax.experimental import pallas as pl
from jax.experimental.pallas import tpu as pltpu
from jax.experimental.pallas import tpu_sc as plsc

STEP = 8
NROWS = 16
SEQ = 2048
D = 512
MAXLEN = SEQ - 1
NOUT = MAXLEN // STEP + 1
TOTAL = NROWS * NOUT
NWORK = 32
CHUNK = TOTAL // NWORK
LANES = 16
NCH = 2
SUB = CHUNK // NCH
PSRC = 16


def _flat_src(p, lens_v):
    b = lax.shift_right_logical(p, 8)
    k = p - b * NOUT
    l = plsc.load_gather(lens_v, [b])
    start = jnp.minimum(l - 1, (STEP - 1) + (l & (STEP - 1)))
    v = start + (k - 1) * STEP
    idx = jnp.where(k == 0, 0, jnp.where(v < l, v, MAXLEN - 1) + 1)
    return b * SEQ + idx


def _body(payload_hbm, lens_hbm, out_hbm, olens_hbm,
          lens_v, idx_v, pidx_v, buf_v, psrc_v, olens_v,
          semg0, semg1, semg2, semg3, semp, semp0, semp1, semp2, semp3):
    wid = lax.axis_index("s") * 2 + lax.axis_index("c")
    pltpu.sync_copy(lens_hbm, lens_v)
    iota = lax.iota(jnp.int32, LANES)
    base = wid * CHUNK
    b = lax.shift_right_logical(wid, 1)
    h = wid & 1

    pidx_v[...] = (b * SEQ + SEQ - PSRC) + iota
    psd = pltpu.async_copy(payload_hbm.at[pidx_v], psrc_v, semp)

    l = jnp.max(plsc.load_gather(lens_v, [iota * 0 + b]))
    bv = jnp.minimum(jnp.maximum(lax.shift_right_logical(l, 3), 1), NOUT - 1)
    fp = jnp.where(h == 0, 1 + jnp.minimum(bv, CHUNK - 1),
                   jnp.clip(bv - (CHUNK - 1), 0, CHUNK))

    for c in range(NCH):
        for g in range(SUB // LANES):
            s = c * SUB + g * LANES + iota
            real = _flat_src(base + s, lens_v)
            dummy = base * (SEQ // NOUT) + s
            idx_v[c, pl.ds(g * LANES, LANES)] = jnp.where(s < fp, real, dummy)

    semg = (semg0, semg1, semg2, semg3)
    gd = [pltpu.async_copy(payload_hbm.at[idx_v.at[t]],
                           buf_v.at[pl.ds(t * SUB, SUB)], semg[t])
          for t in range(NCH)]

    psd.wait()
    padv = [psrc_v[PSRC - 1, pl.ds(i * LANES, LANES)]
            for i in range(D // LANES)]

    dsts = [out_hbm.at[pl.ds(base + t * SUB, SUB)] for t in range(NCH)]
    semput = (semp0, semp1, semp2, semp3)
    pds = [None] * NCH
    for t in range(NCH):
        gd[t].wait()

        def _fill(rr, carry, t=t):
            for i in range(D // LANES):
                buf_v[t * SUB + rr, pl.ds(i * LANES, LANES)] = padv[i]
            return carry

        lo = jnp.clip(fp - t * SUB, 0, SUB)
        lax.fori_loop(lo, SUB, _fill, 0)

        pds[t] = pltpu.async_copy(buf_v.at[pl.ds(t * SUB, SUB)],
                                  dsts[t], semput[t])

    @pl.when(wid == 0)
    def _olens():
        olens_v[...] = jnp.minimum(1, lax.shift_right_logical(lens_v[...], 3))
        pltpu.sync_copy(olens_v, olens_hbm)

    for t in range(NCH):
        pds[t].wait()


_sc_call = pl.kernel(
    _body,
    out_type=(
        jax.ShapeDtypeStruct((TOTAL, D), jnp.float32),
        jax.ShapeDtypeStruct((NROWS,), jnp.int32),
    ),
    mesh=plsc.VectorSubcoreMesh(core_axis_name="c", subcore_axis_name="s"),
    compiler_params=pltpu.CompilerParams(needs_layout_passes=False),
    scratch_types=(
        pltpu.VMEM((NROWS,), jnp.int32),
        pltpu.VMEM((NCH, SUB), jnp.int32),
        pltpu.VMEM((LANES,), jnp.int32),
        pltpu.VMEM((CHUNK, D), jnp.float32),
        pltpu.VMEM((PSRC, D), jnp.float32),
        pltpu.VMEM((NROWS,), jnp.int32),
        pltpu.SemaphoreType.DMA,
        pltpu.SemaphoreType.DMA,
        pltpu.SemaphoreType.DMA,
        pltpu.SemaphoreType.DMA,
        pltpu.SemaphoreType.DMA,
        pltpu.SemaphoreType.DMA,
        pltpu.SemaphoreType.DMA,
        pltpu.SemaphoreType.DMA,
        pltpu.SemaphoreType.DMA,
    ),
)


@jax.jit
def kernel(x_payload, x_seq_lens):
    flat = x_payload.reshape(NROWS * SEQ, D)
    out, out_lens = _sc_call(flat, x_seq_lens.astype(jnp.int32))
    return out.reshape(NROWS, NOUT, D), out_lens


# speedup vs baseline: 1.5199x; 1.0074x over previous
import jax
import jax.numpy as jnp
from jax import lax
from jax.experimental import pallas as pl
from jax.experimental.pallas import tpu as pltpu
from jax.experimental.pallas import tpu_sc as plsc

STEP = 8
NROWS = 16
SEQ = 2048
D = 512
MAXLEN = SEQ - 1
NOUT = MAXLEN // STEP + 1
TOTAL = NROWS * NOUT
NWORK = 32
CHUNK = TOTAL // NWORK
LANES = 16
NCH = 2
SUB = CHUNK // NCH
PSRC = 1


def _flat_src(p, lens_v):
    b = lax.shift_right_logical(p, 8)
    k = p - b * NOUT
    l = plsc.load_gather(lens_v, [b])
    start = jnp.minimum(l - 1, (STEP - 1) + (l & (STEP - 1)))
    v = start + (k - 1) * STEP
    idx = jnp.where(k == 0, 0, jnp.where(v < l, v, MAXLEN - 1) + 1)
    return b * SEQ + idx


def _body(payload_hbm, lens_hbm, out_hbm, olens_hbm,
          lens_v, idx_v, buf_v, psrc_v, olens_v,
          semg0, semg1, semg2, semg3, semp, semp0, semp1, semp2, semp3):
    wid = lax.axis_index("s") * 2 + lax.axis_index("c")
    pltpu.sync_copy(lens_hbm, lens_v)
    iota = lax.iota(jnp.int32, LANES)
    base = wid * CHUNK
    b = lax.shift_right_logical(wid, 1)
    h = wid & 1

    psd = pltpu.async_copy(payload_hbm.at[pl.ds(b * SEQ + SEQ - 1, 1)],
                           psrc_v, semp)

    l = jnp.max(plsc.load_gather(lens_v, [iota * 0 + b]))
    bv = jnp.minimum(jnp.maximum(lax.shift_right_logical(l, 3), 1), NOUT - 1)
    fp = jnp.where(h == 0, 1 + jnp.minimum(bv, CHUNK - 1),
                   jnp.clip(bv - (CHUNK - 1), 0, CHUNK))

    for c in range(NCH):
        for g in range(SUB // LANES):
            s = c * SUB + g * LANES + iota
            real = _flat_src(base + s, lens_v)
            dummy = base * (SEQ // NOUT) + s
            idx_v[c, pl.ds(g * LANES, LANES)] = jnp.where(s < fp, real, dummy)

    semg = (semg0, semg1, semg2, semg3)
    gd = [pltpu.async_copy(payload_hbm.at[idx_v.at[t]],
                           buf_v.at[pl.ds(t * SUB, SUB)], semg[t])
          for t in range(NCH)]

    psd.wait()
    padv = [psrc_v[0, pl.ds(i * LANES, LANES)]
            for i in range(D // LANES)]

    dsts = [out_hbm.at[pl.ds(base + t * SUB, SUB)] for t in range(NCH)]
    semput = (semp0, semp1, semp2, semp3)
    pds = [None] * NCH
    for t in range(NCH):
        gd[t].wait()

        def _fill(rr, carry, t=t):
            for i in range(D // LANES):
                buf_v[t * SUB + rr, pl.ds(i * LANES, LANES)] = padv[i]
            return carry

        lo = jnp.clip(fp - t * SUB, 0, SUB)
        lax.fori_loop(lo, SUB, _fill, 0)

        pds[t] = pltpu.async_copy(buf_v.at[pl.ds(t * SUB, SUB)],
                                  dsts[t], semput[t])

    @pl.when(wid == 0)
    def _olens():
        olens_v[...] = jnp.minimum(1, lax.shift_right_logical(lens_v[...], 3))
        pltpu.sync_copy(olens_v, olens_hbm)

    for t in range(NCH):
        pds[t].wait()


_sc_call = pl.kernel(
    _body,
    out_type=(
        jax.ShapeDtypeStruct((TOTAL, D), jnp.float32),
        jax.ShapeDtypeStruct((NROWS,), jnp.int32),
    ),
    mesh=plsc.VectorSubcoreMesh(core_axis_name="c", subcore_axis_name="s"),
    compiler_params=pltpu.CompilerParams(needs_layout_passes=False),
    scratch_types=(
        pltpu.VMEM((NROWS,), jnp.int32),
        pltpu.VMEM((NCH, SUB), jnp.int32),
        pltpu.VMEM((CHUNK, D), jnp.float32),
        pltpu.VMEM((PSRC, D), jnp.float32),
        pltpu.VMEM((NROWS,), jnp.int32),
        pltpu.SemaphoreType.DMA,
        pltpu.SemaphoreType.DMA,
        pltpu.SemaphoreType.DMA,
        pltpu.SemaphoreType.DMA,
        pltpu.SemaphoreType.DMA,
        pltpu.SemaphoreType.DMA,
        pltpu.SemaphoreType.DMA,
        pltpu.SemaphoreType.DMA,
        pltpu.SemaphoreType.DMA,
    ),
)


@jax.jit
def kernel(x_payload, x_seq_lens):
    flat = x_payload.reshape(NROWS * SEQ, D)
    out, out_lens = _sc_call(flat, x_seq_lens.astype(jnp.int32))
    return out.reshape(NROWS, NOUT, D), out_lens
